# Initial kernel scaffold; baseline (speedup 1.0000x reference)
import functools

import jax
import jax.numpy as jnp
from jax import lax
from jax.experimental import pallas as pl
from jax.experimental.pallas import tpu as pltpu
from jax.experimental.pallas import tpu_sc as plsc

B, C, T, K, GROUPS, WIDTH = 4, 128, 2048, 16, 32, 128
NBLK = 8
RB = T // NBLK
F32 = jnp.float32
NEG = jnp.float32(-3.0e38)


def _block_diag(w):
    g, og, ig = w.shape
    eye = jnp.eye(g, dtype=w.dtype)
    big = jnp.einsum('goi,gh->gohi', w, eye)
    return big.reshape(g * og, g * ig)


def _prep_body(x_ref, w1_ref, b1_ref, w2s_ref, b2_ref, w3_ref, b3_ref,
               at_ref, bmt_ref, sb1_ref, tfull_ref, s1a_ref, s1b_ref):
    xb = x_ref[0]
    t1 = jnp.maximum(
        jnp.dot(w1_ref[...], xb, preferred_element_type=F32) + b1_ref[...], 0.0)
    zcol = jnp.zeros((WIDTH, 1), F32)
    t1m = jnp.concatenate([zcol, t1[:, :-1]], axis=1)
    t1p = jnp.concatenate([t1[:, 1:], zcol], axis=1)
    t2 = (jnp.dot(w2s_ref[0], t1m, preferred_element_type=F32)
          + jnp.dot(w2s_ref[1], t1, preferred_element_type=F32)
          + jnp.dot(w2s_ref[2], t1p, preferred_element_type=F32)
          + b2_ref[...])
    t2 = jnp.maximum(t2, 0.0)
    t3 = jnp.dot(w3_ref[...], t2, preferred_element_type=F32) + b3_ref[...]
    tfull_ref[0] = t3 + xb
    dn = (((0,), (0,)), ((), ()))
    s1a_ref[0] = lax.dot_general(xb, at_ref[...], dn, preferred_element_type=F32)
    s1b_ref[0] = (lax.dot_general(xb, bmt_ref[...], dn,
                                  preferred_element_type=F32) + sb1_ref[...])


def _prep_call(x, w1, b1c, w2s, b2c, w3, b3c, at, bmt, sb1r):
    full2 = lambda a: pl.BlockSpec(a.shape, lambda b: (0,) * a.ndim)
    return pl.pallas_call(
        _prep_body,
        grid=(B,),
        in_specs=[
            pl.BlockSpec((1, C, T), lambda b: (b, 0, 0)),
            full2(w1), full2(b1c), full2(w2s), full2(b2c), full2(w3),
            full2(b3c), full2(at), full2(bmt), full2(sb1r),
        ],
        out_specs=[
            pl.BlockSpec((1, C, T), lambda b: (b, 0, 0)),
            pl.BlockSpec((1, T, WIDTH), lambda b: (b, 0, 0)),
            pl.BlockSpec((1, T, WIDTH), lambda b: (b, 0, 0)),
        ],
        out_shape=[
            jax.ShapeDtypeStruct((B, C, T), F32),
            jax.ShapeDtypeStruct((B, T, WIDTH), F32),
            jax.ShapeDtypeStruct((B, T, WIDTH), F32),
        ],
    )(x, w1, b1c, w2s, b2c, w3, b3c, at, bmt, sb1r)


def _pair_body(xall_ref, xrow_ref, idx_ref):
    b = pl.program_id(0)
    xb = xall_ref[0]
    xr = xrow_ref[0]
    xx = jnp.sum(xb * xb, axis=0, keepdims=True)
    p2 = lax.dot_general(xr, xb, (((0,), (0,)), ((), ())),
                         preferred_element_type=F32,
                         precision=lax.Precision.HIGHEST)
    p = 2.0 * p2 - xx
    cols = lax.broadcasted_iota(jnp.int32, (RB, T), 1)
    picks = []
    for _ in range(K):
        m = jnp.max(p, axis=1, keepdims=True)
        cand = jnp.where(p == m, cols, T)
        j = jnp.min(cand, axis=1, keepdims=True)
        picks.append(j)
        p = jnp.where(cols == j, NEG, p)
    idx_ref[0] = jnp.concatenate(picks, axis=1) + b * T


def _pair_call(x):
    return pl.pallas_call(
        _pair_body,
        grid=(B, NBLK),
        in_specs=[
            pl.BlockSpec((1, C, T), lambda b, nb: (b, 0, 0)),
            pl.BlockSpec((1, C, RB), lambda b, nb: (b, 0, nb)),
        ],
        out_specs=pl.BlockSpec((1, RB, K), lambda b, nb: (b, nb, 0)),
        out_shape=jax.ShapeDtypeStruct((B, T, K), jnp.int32),
    )(x, x)


TOT = B * T * K
CH = 128


def _sc_gather(idx_flat, table):
    info = plsc.get_sparse_core_info()
    nc, ns = info.num_cores, info.num_subcores
    nw = nc * ns
    per_w = TOT // nw
    nchunk = per_w // CH

    @functools.partial(
        pl.kernel,
        mesh=plsc.VectorSubcoreMesh(core_axis_name="c", subcore_axis_name="s"),
        out_type=jax.ShapeDtypeStruct((TOT, WIDTH), F32),
        scratch_types=[
            pltpu.VMEM((CH,), jnp.int32),
            pltpu.VMEM((CH, WIDTH), F32),
            pltpu.SemaphoreType.DMA,
        ],
    )
    def _gather(idx_hbm, table_hbm, out_hbm, idx_v, rows_v, sem):
        wid = lax.axis_index("s") * nc + lax.axis_index("c")
        base = wid * per_w

        def body(c, carry):
            b0 = base + c * CH
            pltpu.sync_copy(idx_hbm.at[pl.ds(b0, CH)], idx_v)
            pltpu.async_copy(table_hbm.at[idx_v], rows_v, sem).wait()
            pltpu.sync_copy(rows_v, out_hbm.at[pl.ds(b0, CH)])
            return carry

        lax.fori_loop(0, nchunk, body, 0)

    return _gather(idx_flat, table)


def _edge_body(g_ref, s1b_ref, tf_ref, w2t_ref, sb2_ref, w3_ref, sb3_ref,
               out_ref):
    g3 = g_ref[...].reshape(RB, K, WIDTH)
    s1b = s1b_ref[0]
    w2t = w2t_ref[...]
    w3 = w3_ref[...]
    m = jnp.full((C, RB), NEG, F32)
    for k in range(K):
        h = jnp.maximum(g3[:, k, :] + s1b, 0.0)
        s2 = jnp.maximum(
            jnp.dot(h, w2t, preferred_element_type=F32) + sb2_ref[...], 0.0)
        s3t = lax.dot_general(w3, s2, (((1,), (1,)), ((), ())),
                              preferred_element_type=F32)
        m = jnp.maximum(m, s3t)
    out_ref[0] = jnp.maximum(tf_ref[0] + m + sb3_ref[...], 0.0)


def _edge_call(g, s1b, tfull, w2t, sb2r, w3, sb3c):
    full2 = lambda a: pl.BlockSpec(a.shape, lambda b, nb: (0,) * a.ndim)
    return pl.pallas_call(
        _edge_body,
        grid=(B, NBLK),
        in_specs=[
            pl.BlockSpec((RB * K, WIDTH), lambda b, nb: (b * NBLK + nb, 0)),
            pl.BlockSpec((1, RB, WIDTH), lambda b, nb: (b, nb, 0)),
            pl.BlockSpec((1, C, RB), lambda b, nb: (b, 0, nb)),
            full2(w2t), full2(sb2r), full2(w3), full2(sb3c),
        ],
        out_specs=pl.BlockSpec((1, C, RB), lambda b, nb: (b, 0, nb)),
        out_shape=jax.ShapeDtypeStruct((B, C, T), F32),
    )(g, s1b, tfull, w2t, sb2r, w3, sb3c)


def kernel(x, tw1, tb1, tw2, tb2, tw3, tb3, sw1, sb1, sw2, sb2, sw3, sb3):
    w1 = tw1[:, :, 0]
    w2s = jnp.stack([_block_diag(tw2[:, :, s].reshape(GROUPS, WIDTH // GROUPS,
                                                      WIDTH // GROUPS))
                     for s in range(3)])
    w3 = tw3[:, :, 0]
    b1c = tb1.reshape(WIDTH, 1)
    b2c = tb2.reshape(WIDTH, 1)
    b3c = tb3.reshape(C, 1)
    at = sw1[:, :C, 0, 0].T
    bmt = sw1[:, C:, 0, 0].T
    sb1r = sb1.reshape(1, WIDTH)
    w2t = _block_diag(sw2[:, :, 0, 0].reshape(GROUPS, WIDTH // GROUPS,
                                              WIDTH // GROUPS)).T
    sb2r = sb2.reshape(1, WIDTH)
    w3e = sw3[:, :, 0, 0]
    sb3c = sb3.reshape(C, 1)

    tfull, s1a, s1b = _prep_call(x, w1, b1c, w2s, b2c, w3, b3c, at, bmt, sb1r)
    idx = _pair_call(x)
    g = _sc_gather(idx.reshape(TOT), s1a.reshape(B * T, WIDTH))
    return _edge_call(g, s1b, tfull, w2t, sb2r, w3e, sb3c)


# speedup vs baseline: 9.4231x; 9.4231x over previous
import functools

import jax
import jax.numpy as jnp
from jax import lax
from jax.experimental import pallas as pl
from jax.experimental.pallas import tpu as pltpu
from jax.experimental.pallas import tpu_sc as plsc

B, C, T, K, GROUPS, WIDTH = 4, 128, 2048, 16, 32, 128
NBLK = 8
RB = T // NBLK
F32 = jnp.float32
NEG = -3.0e38


def _block_diag(w):
    g, og, ig = w.shape
    eye = jnp.eye(g, dtype=w.dtype)
    big = jnp.einsum('goi,gh->gohi', w, eye)
    return big.reshape(g * og, g * ig)


def _prep_body(x_ref, w1_ref, b1_ref, w2s_ref, b2_ref, w3_ref, b3_ref,
               at_ref, bmt_ref, sb1_ref, tfull_ref, s1a_ref, s1b_ref):
    xb = x_ref[0]
    t1 = jnp.maximum(
        jnp.dot(w1_ref[...], xb, preferred_element_type=F32) + b1_ref[...], 0.0)
    zcol = jnp.zeros((WIDTH, 1), F32)
    t1m = jnp.concatenate([zcol, t1[:, :-1]], axis=1)
    t1p = jnp.concatenate([t1[:, 1:], zcol], axis=1)
    t2 = (jnp.dot(w2s_ref[0], t1m, preferred_element_type=F32)
          + jnp.dot(w2s_ref[1], t1, preferred_element_type=F32)
          + jnp.dot(w2s_ref[2], t1p, preferred_element_type=F32)
          + b2_ref[...])
    t2 = jnp.maximum(t2, 0.0)
    t3 = jnp.dot(w3_ref[...], t2, preferred_element_type=F32) + b3_ref[...]
    tfull_ref[0] = t3 + xb
    dn = (((0,), (0,)), ((), ()))
    s1a_ref[0] = lax.dot_general(xb, at_ref[...], dn, preferred_element_type=F32)
    s1b_ref[0] = (lax.dot_general(xb, bmt_ref[...], dn,
                                  preferred_element_type=F32) + sb1_ref[...])


def _prep_call(x, w1, b1c, w2s, b2c, w3, b3c, at, bmt, sb1r):
    full2 = lambda a: pl.BlockSpec(a.shape, lambda b: (0,) * a.ndim)
    return pl.pallas_call(
        _prep_body,
        grid=(B,),
        in_specs=[
            pl.BlockSpec((1, C, T), lambda b: (b, 0, 0)),
            full2(w1), full2(b1c), full2(w2s), full2(b2c), full2(w3),
            full2(b3c), full2(at), full2(bmt), full2(sb1r),
        ],
        out_specs=[
            pl.BlockSpec((1, C, T), lambda b: (b, 0, 0)),
            pl.BlockSpec((1, T, WIDTH), lambda b: (b, 0, 0)),
            pl.BlockSpec((1, T, WIDTH), lambda b: (b, 0, 0)),
        ],
        out_shape=[
            jax.ShapeDtypeStruct((B, C, T), F32),
            jax.ShapeDtypeStruct((B, T, WIDTH), F32),
            jax.ShapeDtypeStruct((B, T, WIDTH), F32),
        ],
    )(x, w1, b1c, w2s, b2c, w3, b3c, at, bmt, sb1r)


def _pair_body(xall_ref, xrow_ref, idx_ref):
    b = pl.program_id(0)
    nb = pl.program_id(1)
    xb = xall_ref[0]
    xr = xrow_ref[0]
    xx = jnp.sum(xb * xb, axis=0, keepdims=True)
    p2 = lax.dot_general(xr, xb, (((0,), (0,)), ((), ())),
                         preferred_element_type=F32)
    inner = -2.0 * p2
    rows = lax.broadcasted_iota(jnp.int32, (RB, RB), 0)
    colsq = lax.broadcasted_iota(jnp.int32, (RB, RB), 1)
    xxslice = jnp.sum(xr * xr, axis=0, keepdims=True)
    xxcol = jnp.sum(jnp.where(rows == colsq, xxslice, 0.0),
                    axis=1, keepdims=True)
    p = (-xxcol - inner) - xx
    cols = lax.broadcasted_iota(jnp.int32, (RB, T), 1)
    picks = []
    for _ in range(K):
        m = jnp.max(p, axis=1, keepdims=True)
        cand = jnp.where(p == m, cols, T)
        j = jnp.min(cand, axis=1, keepdims=True)
        picks.append(j)
        p = jnp.where(cols == j, NEG, p)
    idx_ref[0] = jnp.concatenate(picks, axis=1) + b * T


def _pair_call(x):
    return pl.pallas_call(
        _pair_body,
        grid=(B, NBLK),
        in_specs=[
            pl.BlockSpec((1, C, T), lambda b, nb: (b, 0, 0)),
            pl.BlockSpec((1, C, RB), lambda b, nb: (b, 0, nb)),
        ],
        out_specs=pl.BlockSpec((1, RB, K), lambda b, nb: (b, nb, 0)),
        out_shape=jax.ShapeDtypeStruct((B, T, K), jnp.int32),
    )(x, x)


TOT = B * T * K
CH = 128


def _sc_gather(idx_flat, table):
    info = plsc.get_sparse_core_info()
    nc, ns = info.num_cores, info.num_subcores
    nw = nc * ns
    per_w = TOT // nw
    nchunk = per_w // CH

    @functools.partial(
        pl.kernel,
        mesh=plsc.VectorSubcoreMesh(core_axis_name="c", subcore_axis_name="s"),
        out_type=jax.ShapeDtypeStruct((TOT, WIDTH), F32),
        scratch_types=[
            pltpu.VMEM((CH,), jnp.int32),
            pltpu.VMEM((CH, WIDTH), F32),
            pltpu.SemaphoreType.DMA,
        ],
    )
    def _gather(idx_hbm, table_hbm, out_hbm, idx_v, rows_v, sem):
        wid = lax.axis_index("s") * nc + lax.axis_index("c")
        base = wid * per_w

        def body(c, carry):
            b0 = base + c * CH
            pltpu.sync_copy(idx_hbm.at[pl.ds(b0, CH)], idx_v)
            pltpu.async_copy(table_hbm.at[idx_v], rows_v, sem).wait()
            pltpu.sync_copy(rows_v, out_hbm.at[pl.ds(b0, CH)])
            return carry

        lax.fori_loop(0, nchunk, body, 0)

    return _gather(idx_flat, table)


def _edge_body(g_ref, s1b_ref, tf_ref, w2t_ref, sb2_ref, w3_ref, sb3_ref,
               out_ref):
    g3 = g_ref[...].reshape(RB, K, WIDTH)
    s1b = s1b_ref[0]
    w2t = w2t_ref[...]
    w3 = w3_ref[...]
    m = jnp.full((C, RB), NEG, F32)
    for k in range(K):
        h = jnp.maximum(g3[:, k, :] + s1b, 0.0)
        s2 = jnp.maximum(
            jnp.dot(h, w2t, preferred_element_type=F32) + sb2_ref[...], 0.0)
        s3t = lax.dot_general(w3, s2, (((1,), (1,)), ((), ())),
                              preferred_element_type=F32)
        m = jnp.maximum(m, s3t)
    out_ref[0] = jnp.maximum(tf_ref[0] + m + sb3_ref[...], 0.0)


def _edge_call(g, s1b, tfull, w2t, sb2r, w3, sb3c):
    full2 = lambda a: pl.BlockSpec(a.shape, lambda b, nb: (0,) * a.ndim)
    return pl.pallas_call(
        _edge_body,
        grid=(B, NBLK),
        in_specs=[
            pl.BlockSpec((RB * K, WIDTH), lambda b, nb: (b * NBLK + nb, 0)),
            pl.BlockSpec((1, RB, WIDTH), lambda b, nb: (b, nb, 0)),
            pl.BlockSpec((1, C, RB), lambda b, nb: (b, 0, nb)),
            full2(w2t), full2(sb2r), full2(w3), full2(sb3c),
        ],
        out_specs=pl.BlockSpec((1, C, RB), lambda b, nb: (b, 0, nb)),
        out_shape=jax.ShapeDtypeStruct((B, C, T), F32),
    )(g, s1b, tfull, w2t, sb2r, w3, sb3c)


def kernel(x, tw1, tb1, tw2, tb2, tw3, tb3, sw1, sb1, sw2, sb2, sw3, sb3):
    w1 = tw1[:, :, 0]
    w2s = jnp.stack([_block_diag(tw2[:, :, s].reshape(GROUPS, WIDTH // GROUPS,
                                                      WIDTH // GROUPS))
                     for s in range(3)])
    w3 = tw3[:, :, 0]
    b1c = tb1.reshape(WIDTH, 1)
    b2c = tb2.reshape(WIDTH, 1)
    b3c = tb3.reshape(C, 1)
    at = sw1[:, :C, 0, 0].T
    bmt = sw1[:, C:, 0, 0].T
    sb1r = sb1.reshape(1, WIDTH)
    w2t = _block_diag(sw2[:, :, 0, 0].reshape(GROUPS, WIDTH // GROUPS,
                                              WIDTH // GROUPS)).T
    sb2r = sb2.reshape(1, WIDTH)
    w3e = sw3[:, :, 0, 0]
    sb3c = sb3.reshape(C, 1)

    tfull, s1a, s1b = _prep_call(x, w1, b1c, w2s, b2c, w3, b3c, at, bmt, sb1r)
    idx = _pair_call(x)
    g = _sc_gather(idx.reshape(TOT), s1a.reshape(B * T, WIDTH))
    return _edge_call(g, s1b, tfull, w2t, sb2r, w3e, sb3c)
